# baseline (speedup 1.0000x reference)
import functools

import jax
import jax.numpy as jnp
from jax import lax
from jax.experimental import pallas as pl
from jax.experimental.pallas import tpu as pltpu
from jax.experimental.pallas import tpu_sc as plsc

NC = 2
NS = 16
NW = NC * NS
L = 16

BATCH = 4096
POOL_ROWS = 4096
POOL_LEN = 2048
RPW = BATCH // NW
GPW = RPW // L
WIN = POOL_LEN + 2 * L
WIN_S = 1024 + 2 * L
ROWPAD = POOL_LEN + L


def _body(end_hbm, start_hbm, rpi_hbm, bsz_hbm, occ_hbm, r2t_hbm, out_hbm,
          end_v, start_v, meta_v,
          seg0, seg1, seg2, seg3, seg4, seg5, seg6, seg7,
          row0, row1, row2, row3, row4, row5, row6, row7, bsz_v,
          sseg0, sseg1, sseg2, sseg3, sseg4, sseg5, sseg6, sseg7,
          srow0, srow1, srow2, srow3, srow4, srow5, srow6, srow7,
          sout0, sout1, sout2, sout3, sout4, sout5, sout6, sout7):
    cid = lax.axis_index("c")
    sid = lax.axis_index("s")
    wid = sid * NC + cid
    g0 = wid * GPW

    pltpu.sync_copy(end_hbm, end_v)
    pltpu.sync_copy(start_hbm, start_v)
    pltpu.sync_copy(bsz_hbm, bsz_v)
    pltpu.sync_copy(rpi_hbm.at[pl.ds(pl.multiple_of(wid * RPW, RPW), RPW)],
                    meta_v.at[pl.ds(4 * RPW, RPW)])

    bsz = bsz_v[pl.ds(0, L)]
    iota = lax.iota(jnp.int32, L)

    def p1(g, base):
        gl = g * jnp.int32(L)
        e = end_v[pl.ds(gl, L)]
        s = start_v[pl.ds(gl, L)]
        ln = jnp.where(iota + gl < bsz, e - s, jnp.int32(0))
        cs = plsc.cumsum(ln)

        g0i = g0.astype(jnp.int32)

        @pl.when(jnp.logical_and(g >= g0i, g < g0i + jnp.int32(GPW)))
        def _():
            off = (g - g0i) * jnp.int32(4 * L)
            idx4 = iota * jnp.int32(4) + off
            r = meta_v[pl.ds(jnp.int32(4 * RPW) + (g - g0i) * jnp.int32(L), L)]
            plsc.store_scatter(meta_v, [idx4], base + cs - ln)
            plsc.store_scatter(meta_v, [idx4 + jnp.int32(1)], ln)
            plsc.store_scatter(meta_v, [idx4 + jnp.int32(2)], s)
            plsc.store_scatter(meta_v, [idx4 + jnp.int32(3)], r)

        return base + cs[L - 1]

    lax.fori_loop(jnp.int32(0), jnp.int32(BATCH // L), p1, jnp.int32(0),
                  unroll=False)

    segs = (seg0, seg1, seg2, seg3, seg4, seg5, seg6, seg7)
    rows = (row0, row1, row2, row3, row4, row5, row6, row7)
    ssegs = (sseg0, sseg1, sseg2, sseg3, sseg4, sseg5, sseg6, sseg7)
    srows = (srow0, srow1, srow2, srow3, srow4, srow5, srow6, srow7)
    souts = (sout0, sout1, sout2, sout3, sout4, sout5, sout6, sout7)

    def fetch(i):
        m = meta_v[pl.ds(i * jnp.int32(4), L)]
        kv, ln, st, dst = m[0], m[1], m[2], m[3]
        a0 = pl.multiple_of((kv >> 4) << 4, L)
        return ln, st, dst, a0, kv - a0

    def seg_copy(v, p, wait):
        ln, _, _, a0, _ = v
        small = ln <= jnp.int32(WIN_S - 2 * L)

        @pl.when(small)
        def _():
            c = pltpu.make_async_copy(occ_hbm.at[pl.ds(a0, WIN_S)],
                                      segs[p].at[pl.ds(0, WIN_S)], ssegs[p])
            c.wait() if wait else c.start()

        @pl.when(jnp.logical_not(small))
        def _():
            c = pltpu.make_async_copy(occ_hbm.at[pl.ds(a0, WIN)], segs[p],
                                      ssegs[p])
            c.wait() if wait else c.start()

    def row_copy(v, p, wait):
        ln, st, dst, _, _ = v
        sel = jnp.where(st == jnp.int32(0), ln >> 9, jnp.int32(0))
        for k in range(4):
            @pl.when(sel == jnp.int32(k))
            def _(_k=k):
                a, sz = _k * 512, POOL_LEN - _k * 512
                c = pltpu.make_async_copy(
                    r2t_hbm.at[dst, pl.ds(a, sz)],
                    rows[p].at[pl.ds(a, sz)], srows[p])
                c.wait() if wait else c.start()

    def start_in(i, p):
        v = fetch(i)
        seg_copy(v, p, False)
        row_copy(v, p, False)

    def wait_out(p):
        pltpu.make_async_copy(rows[p].at[pl.ds(0, POOL_LEN)],
                              out_hbm.at[jnp.int32(0)], souts[p]).wait()

    for j in range(4):
        start_in(jnp.int32(j), j)
    NB = 8
    DEPTH = 4

    def body2(ih, carry):
        i8 = ih * jnp.int32(NB)
        for b in range(NB):
            i = i8 + jnp.int32(b)
            p, q = b, (b + DEPTH) % NB
            if b < DEPTH:
                @pl.when(i8 > jnp.int32(0))
                def _(_q=q):
                    wait_out(_q)
                start_in(i + jnp.int32(DEPTH), q)
            else:
                wait_out(q)

                @pl.when(i8 < jnp.int32(RPW - NB))
                def _(_q=q, _i=i):
                    start_in(_i + jnp.int32(DEPTH), _q)

            v = fetch(i)
            seg_copy(v, p, True)
            row_copy(v, p, True)

            ln, st, dst, _, sh = v
            nfull = ln >> 4
            n8 = nfull >> 3

            def m8(c, cc, _p=p, _st=st, _sh=sh):
                cl = c * jnp.int32(8 * L)
                for u in range(8):
                    rows[_p][pl.ds(_st + cl + u * L, L)] = (
                        segs[_p][pl.ds(_sh + cl + u * L, L)])
                return cc

            lax.fori_loop(jnp.int32(0), n8, m8, jnp.int32(0), unroll=False)

            def m1(c, cc, _p=p, _st=st, _sh=sh, _n8=n8):
                cl = _n8 * jnp.int32(8 * L) + c * jnp.int32(L)
                rows[_p][pl.ds(_st + cl, L)] = segs[_p][pl.ds(_sh + cl, L)]
                return cc

            lax.fori_loop(jnp.int32(0), nfull & jnp.int32(7), m1, jnp.int32(0),
                          unroll=False)
            rem = ln & jnp.int32(L - 1)

            @pl.when(rem > jnp.int32(0))
            def _(_p=p, _st=st, _sh=sh, _nf=nfull, _rem=rem):
                cl = _nf * jnp.int32(L)
                vals = segs[_p][pl.ds(_sh + cl, L)]
                old = rows[_p][pl.ds(_st + cl, L)]
                rows[_p][pl.ds(_st + cl, L)] = jnp.where(iota < _rem, vals,
                                                         old)

            pltpu.async_copy(rows[p].at[pl.ds(0, POOL_LEN)], out_hbm.at[dst],
                             souts[p])
        return carry

    lax.fori_loop(jnp.int32(0), jnp.int32(RPW // NB), body2, jnp.int32(0),
                  unroll=False)
    for j in range(4, 8):
        wait_out(j)


@jax.jit
def _run(end32, start32, rpi32, bsz_arr, occ, r2t):
    kfn = pl.kernel(
        _body,
        out_type=jax.ShapeDtypeStruct((POOL_ROWS, POOL_LEN), jnp.float32),
        mesh=plsc.VectorSubcoreMesh(core_axis_name="c", subcore_axis_name="s",
                                    num_cores=NC, num_subcores=NS),
        scratch_types=[
            pltpu.VMEM((BATCH,), jnp.int32),
            pltpu.VMEM((BATCH,), jnp.int32),
            pltpu.VMEM((5 * RPW + L,), jnp.int32),
            *([pltpu.VMEM((WIN,), jnp.float32)] * 8),
            *([pltpu.VMEM((ROWPAD,), jnp.float32)] * 8),
            pltpu.VMEM((L,), jnp.int32),
            *([pltpu.SemaphoreType.DMA] * 24),
        ],
        compiler_params=pltpu.CompilerParams(needs_layout_passes=False),
    )
    return kfn(end32, start32, rpi32, bsz_arr, occ, r2t)


def kernel(req_pool_indices, req_to_token, start_offset, end_offset,
           out_cache_loc, batch_size):
    end32 = end_offset.astype(jnp.int32)
    start32 = start_offset.astype(jnp.int32)
    rpi32 = req_pool_indices.astype(jnp.int32)
    bsz_arr = jnp.full((L,), batch_size, dtype=jnp.int32)
    occ = out_cache_loc.astype(jnp.float32)
    r2t = req_to_token.astype(jnp.float32)
    return _run(end32, start32, rpi32, bsz_arr, occ, r2t)


# speedup vs baseline: 1.1994x; 1.1994x over previous
import functools

import jax
import jax.numpy as jnp
from jax import lax
from jax.experimental import pallas as pl
from jax.experimental.pallas import tpu as pltpu
from jax.experimental.pallas import tpu_sc as plsc

NC = 2
NS = 16
NW = NC * NS
L = 16

BATCH = 4096
POOL_ROWS = 4096
POOL_LEN = 2048
RPW = BATCH // NW
GPW = RPW // L
WIN = POOL_LEN + 2 * L
WIN_S = 1024 + 2 * L
ROWPAD = POOL_LEN + L


def _body(end_hbm, start_hbm, rpi_hbm, bsz_hbm, occ_hbm, r2t_hbm, out_hbm,
          end_v, start_v, meta_v,
          seg0, seg1, seg2, seg3, seg4, seg5, seg6, seg7,
          row0, row1, row2, row3, row4, row5, row6, row7, bsz_v,
          sseg0, sseg1, sseg2, sseg3, sseg4, sseg5, sseg6, sseg7,
          srow0, srow1, srow2, srow3, srow4, srow5, srow6, srow7,
          sout0, sout1, sout2, sout3, sout4, sout5, sout6, sout7):
    cid = lax.axis_index("c")
    sid = lax.axis_index("s")
    wid = sid * NC + cid
    g0 = wid * GPW

    pltpu.sync_copy(end_hbm, end_v)
    pltpu.sync_copy(start_hbm, start_v)
    pltpu.sync_copy(bsz_hbm, bsz_v)
    pltpu.sync_copy(rpi_hbm.at[pl.ds(pl.multiple_of(wid * RPW, RPW), RPW)],
                    meta_v.at[pl.ds(3 * RPW, RPW)])

    bsz = bsz_v[pl.ds(0, L)]
    iota = lax.iota(jnp.int32, L)

    def p1(g, base):
        gl = g * jnp.int32(L)
        e = end_v[pl.ds(gl, L)]
        s = start_v[pl.ds(gl, L)]
        ln = jnp.where(iota + gl < bsz, e - s, jnp.int32(0))
        cs = plsc.cumsum(ln)

        g0i = g0.astype(jnp.int32)

        @pl.when(jnp.logical_and(g >= g0i, g < g0i + jnp.int32(GPW)))
        def _():
            off = (g - g0i) * jnp.int32(L)
            meta_v[pl.ds(off, L)] = base + cs - ln
            meta_v[pl.ds(RPW + off, L)] = ln
            meta_v[pl.ds(2 * RPW + off, L)] = s

        return base + cs[L - 1]

    lax.fori_loop(jnp.int32(0), jnp.int32(BATCH // L), p1, jnp.int32(0),
                  unroll=False)

    segs = (seg0, seg1, seg2, seg3, seg4, seg5, seg6, seg7)
    rows = (row0, row1, row2, row3, row4, row5, row6, row7)
    ssegs = (sseg0, sseg1, sseg2, sseg3, sseg4, sseg5, sseg6, sseg7)
    srows = (srow0, srow1, srow2, srow3, srow4, srow5, srow6, srow7)
    souts = (sout0, sout1, sout2, sout3, sout4, sout5, sout6, sout7)

    def fetch(i):
        kv = meta_v[pl.ds(i, L)][0]
        ln = meta_v[pl.ds(jnp.int32(RPW) + i, L)][0]
        st = meta_v[pl.ds(jnp.int32(2 * RPW) + i, L)][0]
        dst = meta_v[pl.ds(jnp.int32(3 * RPW) + i, L)][0]
        a0 = pl.multiple_of((kv >> 4) << 4, L)
        return ln, st, dst, a0, kv - a0

    def seg_copy(v, p, wait):
        ln, _, _, a0, _ = v
        small = ln <= jnp.int32(WIN_S - 2 * L)

        @pl.when(small)
        def _():
            c = pltpu.make_async_copy(occ_hbm.at[pl.ds(a0, WIN_S)],
                                      segs[p].at[pl.ds(0, WIN_S)], ssegs[p])
            c.wait() if wait else c.start()

        @pl.when(jnp.logical_not(small))
        def _():
            c = pltpu.make_async_copy(occ_hbm.at[pl.ds(a0, WIN)], segs[p],
                                      ssegs[p])
            c.wait() if wait else c.start()

    def row_copy(v, p, wait):
        ln, st, dst, _, _ = v
        sel = jnp.where(st == jnp.int32(0), ln >> 9, jnp.int32(0))
        for k in range(4):
            @pl.when(sel == jnp.int32(k))
            def _(_k=k):
                a, sz = _k * 512, POOL_LEN - _k * 512
                c = pltpu.make_async_copy(
                    r2t_hbm.at[dst, pl.ds(a, sz)],
                    rows[p].at[pl.ds(a, sz)], srows[p])
                c.wait() if wait else c.start()

    def start_in(i, p):
        v = fetch(i)
        seg_copy(v, p, False)
        row_copy(v, p, False)

    def wait_out(p):
        pltpu.make_async_copy(rows[p].at[pl.ds(0, POOL_LEN)],
                              out_hbm.at[jnp.int32(0)], souts[p]).wait()

    for j in range(4):
        start_in(jnp.int32(j), j)
    NB = 8
    DEPTH = 4

    def body2(ih, carry):
        i8 = ih * jnp.int32(NB)
        for b in range(NB):
            i = i8 + jnp.int32(b)
            p, q = b, (b + DEPTH) % NB
            if b < DEPTH:
                @pl.when(i8 > jnp.int32(0))
                def _(_q=q):
                    wait_out(_q)
                start_in(i + jnp.int32(DEPTH), q)
            else:
                wait_out(q)

                @pl.when(i8 < jnp.int32(RPW - NB))
                def _(_q=q, _i=i):
                    start_in(_i + jnp.int32(DEPTH), _q)

            v = fetch(i)
            seg_copy(v, p, True)
            row_copy(v, p, True)

            ln, st, dst, _, sh = v
            nfull = ln >> 4
            n4 = nfull >> 2

            def m4(c, cc, _p=p, _st=st, _sh=sh):
                cl = c * jnp.int32(4 * L)
                for u in range(4):
                    rows[_p][pl.ds(_st + cl + u * L, L)] = (
                        segs[_p][pl.ds(_sh + cl + u * L, L)])
                return cc

            lax.fori_loop(jnp.int32(0), n4, m4, jnp.int32(0), unroll=False)

            def m1(c, cc, _p=p, _st=st, _sh=sh, _n4=n4):
                cl = _n4 * jnp.int32(4 * L) + c * jnp.int32(L)
                rows[_p][pl.ds(_st + cl, L)] = segs[_p][pl.ds(_sh + cl, L)]
                return cc

            lax.fori_loop(jnp.int32(0), nfull & jnp.int32(3), m1, jnp.int32(0),
                          unroll=False)
            rem = ln & jnp.int32(L - 1)

            @pl.when(rem > jnp.int32(0))
            def _(_p=p, _st=st, _sh=sh, _nf=nfull, _rem=rem):
                cl = _nf * jnp.int32(L)
                vals = segs[_p][pl.ds(_sh + cl, L)]
                old = rows[_p][pl.ds(_st + cl, L)]
                rows[_p][pl.ds(_st + cl, L)] = jnp.where(iota < _rem, vals,
                                                         old)

            pltpu.async_copy(rows[p].at[pl.ds(0, POOL_LEN)], out_hbm.at[dst],
                             souts[p])
        return carry

    lax.fori_loop(jnp.int32(0), jnp.int32(RPW // NB), body2, jnp.int32(0),
                  unroll=False)
    for j in range(4, 8):
        wait_out(j)


@jax.jit
def _run(end32, start32, rpi32, bsz_arr, occ, r2t):
    kfn = pl.kernel(
        _body,
        out_type=jax.ShapeDtypeStruct((POOL_ROWS, POOL_LEN), jnp.float32),
        mesh=plsc.VectorSubcoreMesh(core_axis_name="c", subcore_axis_name="s",
                                    num_cores=NC, num_subcores=NS),
        scratch_types=[
            pltpu.VMEM((BATCH,), jnp.int32),
            pltpu.VMEM((BATCH,), jnp.int32),
            pltpu.VMEM((4 * RPW + L,), jnp.int32),
            *([pltpu.VMEM((WIN,), jnp.float32)] * 8),
            *([pltpu.VMEM((ROWPAD,), jnp.float32)] * 8),
            pltpu.VMEM((L,), jnp.int32),
            *([pltpu.SemaphoreType.DMA] * 24),
        ],
        compiler_params=pltpu.CompilerParams(needs_layout_passes=False),
    )
    return kfn(end32, start32, rpi32, bsz_arr, occ, r2t)


def kernel(req_pool_indices, req_to_token, start_offset, end_offset,
           out_cache_loc, batch_size):
    end32 = end_offset.astype(jnp.int32)
    start32 = start_offset.astype(jnp.int32)
    rpi32 = req_pool_indices.astype(jnp.int32)
    bsz_arr = jnp.full((L,), batch_size, dtype=jnp.int32)
    occ = out_cache_loc.astype(jnp.float32)
    r2t = req_to_token.astype(jnp.float32)
    return _run(end32, start32, rpi32, bsz_arr, occ, r2t)
